# Initial kernel scaffold; baseline (speedup 1.0000x reference)
import jax
import jax.numpy as jnp
from jax.experimental import pallas as pl


def kernel(up_logits, down_logits, y_true, masks):
    raise NotImplementedError("write your pallas kernel here")


# speedup vs baseline: 3.3621x; 3.3621x over previous
import jax
import jax.numpy as jnp
from jax.experimental import pallas as pl
from jax.experimental.pallas import tpu as pltpu

_TOP_K = 10
_RANKING_WEIGHT = 0.3
_UP_WEIGHT = 1.0
_DOWN_WEIGHT = 0.5


def _body(up_ref, dn_ref, yt_ref, out_ref):
    i = pl.program_id(0)
    ul = up_ref[...]
    dl = dn_ref[...]
    y = yt_ref[...]
    k = min(_TOP_K, y.shape[1])
    neg_inf = jnp.float32(float("-inf"))
    pos_inf = jnp.float32(float("inf"))

    f_ul = jnp.maximum(ul, 0.0) + jnp.log1p(jnp.exp(-jnp.abs(ul)))
    f_dl = jnp.maximum(dl, 0.0) + jnp.log1p(jnp.exp(-jnp.abs(dl)))
    s_total = _UP_WEIGHT * jnp.sum(f_ul) + _DOWN_WEIGHT * jnp.sum(f_dl)

    m_y = jnp.max(y, axis=1, keepdims=True)
    e_y = jnp.exp(y - m_y)
    s_y = jnp.sum(e_y, axis=1, keepdims=True)
    sum_ey_y = jnp.sum(e_y * y, axis=1, keepdims=True)
    sum_ey_u = jnp.sum(e_y * ul, axis=1, keepdims=True)
    lse_y = m_y + jnp.log(s_y)
    m_u = jnp.max(ul, axis=1, keepdims=True)
    s_u = jnp.sum(jnp.exp(ul - m_u), axis=1, keepdims=True)
    lse_u = m_u + jnp.log(s_u)
    kl_rows = (sum_ey_y - sum_ey_u) / s_y - lse_y + lse_u

    t_hi = m_y
    for _ in range(k - 1):
        t_hi = jnp.max(jnp.where(y < t_hi, y, neg_inf), axis=1, keepdims=True)
    t_lo = jnp.min(y, axis=1, keepdims=True)
    for _ in range(k - 1):
        t_lo = jnp.min(jnp.where(y > t_lo, y, pos_inf), axis=1, keepdims=True)

    kf = jnp.float32(k)
    one = jnp.float32(1.0)
    gt = y > t_hi
    eq_hi = y == t_hi
    cnt_gt = jnp.sum(gt.astype(jnp.float32), axis=1, keepdims=True)
    cnt_eq_hi = jnp.sum(eq_hi.astype(jnp.float32), axis=1, keepdims=True)
    coef_hi = (kf - cnt_gt) / jnp.maximum(cnt_eq_hi, one)
    t_up = (jnp.sum(jnp.where(gt, ul, 0.0), axis=1, keepdims=True)
            + coef_hi * jnp.sum(jnp.where(eq_hi, ul, 0.0), axis=1, keepdims=True))

    lt = y < t_lo
    eq_lo = y == t_lo
    cnt_lt = jnp.sum(lt.astype(jnp.float32), axis=1, keepdims=True)
    cnt_eq_lo = jnp.sum(eq_lo.astype(jnp.float32), axis=1, keepdims=True)
    coef_lo = (kf - cnt_lt) / jnp.maximum(cnt_eq_lo, one)
    t_dn = (jnp.sum(jnp.where(lt, dl, 0.0), axis=1, keepdims=True)
            + coef_lo * jnp.sum(jnp.where(eq_lo, dl, 0.0), axis=1, keepdims=True))

    block_sum = (s_total
                 - _UP_WEIGHT * jnp.sum(t_up)
                 - _DOWN_WEIGHT * jnp.sum(t_dn)
                 + _RANKING_WEIGHT * jnp.sum(kl_rows))

    @pl.when(i == 0)
    def _init():
        out_ref[0, 0] = block_sum

    @pl.when(i != 0)
    def _acc():
        out_ref[0, 0] += block_sum


def kernel(up_logits, down_logits, y_true, masks):
    del masks
    B, N = up_logits.shape
    R = 64
    assert B % R == 0
    out = pl.pallas_call(
        _body,
        grid=(B // R,),
        in_specs=[pl.BlockSpec((R, N), lambda i: (i, 0))] * 3,
        out_specs=pl.BlockSpec((1, 1), lambda i: (0, 0),
                               memory_space=pltpu.SMEM),
        out_shape=jax.ShapeDtypeStruct((1, 1), jnp.float32),
    )(up_logits, down_logits, y_true)
    return (out[0, 0] / jnp.float32(B * N)).astype(jnp.float32)


# speedup vs baseline: 3.5655x; 1.0605x over previous
import jax
import jax.numpy as jnp
from jax.experimental import pallas as pl
from jax.experimental.pallas import tpu as pltpu

_TOP_K = 10
_RANKING_WEIGHT = 0.3
_UP_WEIGHT = 1.0
_DOWN_WEIGHT = 0.5
_LANE = 128
_FOLD_DEPTH = 4

_NEG_INF = float("-inf")
_POS_INF = float("inf")


def _slices(y, largest):
    R, N = y.shape
    n_full = N // _LANE
    out = [y[:, j * _LANE:(j + 1) * _LANE] for j in range(n_full)]
    rem = N - n_full * _LANE
    if rem:
        pad = jnp.full((R, _LANE - rem), _NEG_INF if largest else _POS_INF,
                       dtype=y.dtype)
        out.append(jnp.concatenate([y[:, n_full * _LANE:], pad], axis=1))
    return out


def _fold(slices, depth, largest):
    sent = _NEG_INF if largest else _POS_INF
    acc = [jnp.full_like(slices[0], sent) for _ in range(depth)]
    for x in slices:
        for d in range(depth):
            if largest:
                hi = jnp.maximum(acc[d], x)
                x = jnp.minimum(acc[d], x)
            else:
                hi = jnp.minimum(acc[d], x)
                x = jnp.maximum(acc[d], x)
            acc[d] = hi
    return jnp.concatenate(acc, axis=1)


def _rank_k(cand, k, largest):
    sent = _NEG_INF if largest else _POS_INF
    red = (lambda a: jnp.max(a, axis=1, keepdims=True)) if largest else \
          (lambda a: jnp.min(a, axis=1, keepdims=True))
    R = cand.shape[0]
    kf = jnp.float32(k)
    cum = jnp.zeros((R, 1), jnp.float32)
    t = jnp.zeros((R, 1), jnp.float32)
    m = None
    for i in range(k):
        if i == 0:
            m = red(cand)
        else:
            if largest:
                m = red(jnp.where(cand < m, cand, sent))
            else:
                m = red(jnp.where(cand > m, cand, sent))
        c = jnp.sum((cand == m).astype(jnp.float32), axis=1, keepdims=True)
        crossed = jnp.logical_and(cum < kf, cum + c >= kf)
        t = t + jnp.where(crossed, m, 0.0)
        cum = cum + c
    return t


def _masked_sum(y, t, logits, k, largest):
    if largest:
        strict = y > t
    else:
        strict = y < t
    eq = y == t
    cnt_s = jnp.sum(strict.astype(jnp.float32), axis=1, keepdims=True)
    cnt_e = jnp.sum(eq.astype(jnp.float32), axis=1, keepdims=True)
    coef = jnp.clip((jnp.float32(k) - cnt_s) / jnp.maximum(cnt_e, 1.0),
                    0.0, 1.0)
    w = jnp.where(strict, logits, 0.0) + coef * jnp.where(eq, logits, 0.0)
    return jnp.sum(w, axis=1, keepdims=True)


def _body(up_ref, dn_ref, yt_ref, out_ref, thi_ref, tlo_ref):
    i = pl.program_id(0)
    ul = up_ref[...]
    dl = dn_ref[...]
    y = yt_ref[...]
    R, N = y.shape
    k = min(_TOP_K, N)

    f_ul = jnp.maximum(ul, 0.0) + jnp.log1p(jnp.exp(-jnp.abs(ul)))
    f_dl = jnp.maximum(dl, 0.0) + jnp.log1p(jnp.exp(-jnp.abs(dl)))
    s_total = _UP_WEIGHT * jnp.sum(f_ul) + _DOWN_WEIGHT * jnp.sum(f_dl)

    m_y = jnp.max(y, axis=1, keepdims=True)
    e_y = jnp.exp(y - m_y)
    s_y = jnp.sum(e_y, axis=1, keepdims=True)
    sum_ey_y = jnp.sum(e_y * y, axis=1, keepdims=True)
    sum_ey_u = jnp.sum(e_y * ul, axis=1, keepdims=True)
    lse_y = m_y + jnp.log(s_y)
    m_u = jnp.max(ul, axis=1, keepdims=True)
    s_u = jnp.sum(jnp.exp(ul - m_u), axis=1, keepdims=True)
    lse_u = m_u + jnp.log(s_u)
    kl_rows = (sum_ey_y - sum_ey_u) / s_y - lse_y + lse_u

    sl_hi = _slices(y, largest=True)
    sl_lo = _slices(y, largest=False)

    cand_hi = _fold(sl_hi, _FOLD_DEPTH, largest=True)
    t_hi = _rank_k(cand_hi, k, largest=True)
    thi_ref[...] = t_hi
    m4_hi = cand_hi[:, (_FOLD_DEPTH - 1) * _LANE:]
    bad_hi = jnp.max(jnp.where(m4_hi > t_hi, 1.0, 0.0)) > 0.5

    cand_lo = _fold(sl_lo, _FOLD_DEPTH, largest=False)
    t_lo = _rank_k(cand_lo, k, largest=False)
    tlo_ref[...] = t_lo
    m4_lo = cand_lo[:, (_FOLD_DEPTH - 1) * _LANE:]
    bad_lo = jnp.max(jnp.where(m4_lo < t_lo, 1.0, 0.0)) > 0.5

    @pl.when(bad_hi)
    def _fallback_hi():
        thi_ref[...] = _rank_k(_fold(sl_hi, k, largest=True), k,
                               largest=True)

    @pl.when(bad_lo)
    def _fallback_lo():
        tlo_ref[...] = _rank_k(_fold(sl_lo, k, largest=False), k,
                               largest=False)

    t_up = _masked_sum(y, thi_ref[...], ul, k, largest=True)
    t_dn = _masked_sum(y, tlo_ref[...], dl, k, largest=False)

    block_sum = (s_total
                 - _UP_WEIGHT * jnp.sum(t_up)
                 - _DOWN_WEIGHT * jnp.sum(t_dn)
                 + _RANKING_WEIGHT * jnp.sum(kl_rows))

    @pl.when(i == 0)
    def _init():
        out_ref[0, 0] = block_sum

    @pl.when(i != 0)
    def _acc():
        out_ref[0, 0] += block_sum


def kernel(up_logits, down_logits, y_true, masks):
    del masks
    B, N = up_logits.shape
    R = 64
    assert B % R == 0
    out = pl.pallas_call(
        _body,
        grid=(B // R,),
        in_specs=[pl.BlockSpec((R, N), lambda i: (i, 0))] * 3,
        out_specs=pl.BlockSpec((1, 1), lambda i: (0, 0),
                               memory_space=pltpu.SMEM),
        out_shape=jax.ShapeDtypeStruct((1, 1), jnp.float32),
        scratch_shapes=[pltpu.VMEM((R, 1), jnp.float32),
                        pltpu.VMEM((R, 1), jnp.float32)],
    )(up_logits, down_logits, y_true)
    return (out[0, 0] / jnp.float32(B * N)).astype(jnp.float32)


# speedup vs baseline: 4.5408x; 1.2735x over previous
import jax
import jax.numpy as jnp
from jax.experimental import pallas as pl
from jax.experimental.pallas import tpu as pltpu

_TOP_K = 10
_RANKING_WEIGHT = 0.3
_UP_WEIGHT = 1.0
_DOWN_WEIGHT = 0.5
_LANE = 128
_DEPTH = 4
_GR = 8

_NEG_INF = float("-inf")
_POS_INF = float("inf")
_NEG_BIG = -1e30
_POS_BIG = 1e30

_LOG2E = 1.4426950408889634
_LN2 = 0.6931471805599453


def _rank_k(cand, k, largest):
    sent = _NEG_INF if largest else _POS_INF
    red = (lambda a: jnp.max(a, axis=1, keepdims=True)) if largest else \
          (lambda a: jnp.min(a, axis=1, keepdims=True))
    rows = cand.shape[0]
    kf = jnp.float32(k)
    cum = jnp.zeros((rows, 1), jnp.float32)
    t = jnp.zeros((rows, 1), jnp.float32)
    m = None
    for i in range(k):
        if i == 0:
            m = red(cand)
        else:
            if largest:
                m = red(jnp.where(cand < m, cand, sent))
            else:
                m = red(jnp.where(cand > m, cand, sent))
        c = jnp.sum((cand == m).astype(jnp.float32), axis=1, keepdims=True)
        crossed = jnp.logical_and(cum < kf, cum + c >= kf)
        t = t + jnp.where(crossed, m, 0.0)
        cum = cum + c
    return t


def _coef(vals, t, k, largest):
    strict = (vals > t) if largest else (vals < t)
    eq = vals == t
    cnt_s = jnp.sum(strict.astype(jnp.float32), axis=1, keepdims=True)
    cnt_e = jnp.sum(eq.astype(jnp.float32), axis=1, keepdims=True)
    return jnp.clip((jnp.float32(k) - cnt_s) / jnp.maximum(cnt_e, 1.0),
                    0.0, 1.0)


def _fold_full(y, depth, largest):
    rows, N = y.shape
    n_full = N // _LANE
    sent = _NEG_INF if largest else _POS_INF
    pad_v = _NEG_BIG if largest else _POS_BIG
    acc = [jnp.full((rows, _LANE), sent, jnp.float32) for _ in range(depth)]
    sls = [y[:, j * _LANE:(j + 1) * _LANE] for j in range(n_full)]
    rem = N - n_full * _LANE
    if rem:
        pad = jnp.full((rows, _LANE - rem), pad_v, jnp.float32)
        sls.append(jnp.concatenate([y[:, n_full * _LANE:], pad], axis=1))
    for x in sls:
        for d in range(depth):
            if largest:
                hi = jnp.maximum(acc[d], x)
                x = jnp.minimum(acc[d], x)
            else:
                hi = jnp.minimum(acc[d], x)
                x = jnp.maximum(acc[d], x)
            acc[d] = hi
    return jnp.concatenate(acc, axis=1)


def _body(up_ref, dn_ref, yt_ref, out_ref, thi_ref, tlo_ref,
          chi_ref, clo_ref):
    i = pl.program_id(0)
    R, N = yt_ref.shape
    k = min(_TOP_K, N)
    n_full = N // _LANE
    rem = N - n_full * _LANE
    n_groups = R // _GR
    log2e = jnp.float32(_LOG2E)
    ln2 = jnp.float32(_LN2)

    def load(ref, g, j):
        r0 = g * _GR
        if j < n_full:
            return ref[r0:r0 + _GR, j * _LANE:(j + 1) * _LANE]
        return ref[r0:r0 + _GR, n_full * _LANE:]

    def padded(x, pad_v):
        if x.shape[1] == _LANE:
            return x
        return jnp.concatenate(
            [x, jnp.full((x.shape[0], _LANE - x.shape[1]), pad_v,
                         jnp.float32)], axis=1)

    n_slices = n_full + (1 if rem else 0)

    total = jnp.float32(0.0)
    bad_hi_any = jnp.float32(0.0)
    bad_lo_any = jnp.float32(0.0)
    kl_parts = []

    for g in range(n_groups):
        zeros = jnp.zeros((_GR, _LANE), jnp.float32)
        f_acc = zeros
        sy_acc = zeros
        eyy_acc = zeros
        eyu_acc = zeros
        su_acc = zeros
        acc_hi = [jnp.full((_GR, _LANE), _NEG_INF, jnp.float32)
                  for _ in range(_DEPTH)]
        acc_lo = [jnp.full((_GR, _LANE), _POS_INF, jnp.float32)
                  for _ in range(_DEPTH)]
        for j in range(n_slices):
            ysl = padded(load(yt_ref, g, j), _NEG_BIG)
            usl = padded(load(up_ref, g, j), _NEG_BIG)
            dsl = padded(load(dn_ref, g, j), _NEG_BIG)
            p_u = jnp.exp2(usl * log2e)
            p_d = jnp.exp2(dsl * log2e)
            e_y = jnp.exp2(ysl * log2e)
            su_acc = su_acc + p_u
            f_acc = (f_acc + jnp.log2(1.0 + p_u)
                     + jnp.float32(_DOWN_WEIGHT) * jnp.log2(1.0 + p_d))
            sy_acc = sy_acc + e_y
            eyy_acc = eyy_acc + e_y * ysl
            eyu_acc = eyu_acc + e_y * usl
            xh = ysl
            xl = ysl if j < n_full else padded(load(yt_ref, g, j), _POS_BIG)
            for d in range(_DEPTH):
                hi = jnp.maximum(acc_hi[d], xh)
                xh = jnp.minimum(acc_hi[d], xh)
                acc_hi[d] = hi
                lo = jnp.minimum(acc_lo[d], xl)
                xl = jnp.maximum(acc_lo[d], xl)
                acc_lo[d] = lo

        s_y = jnp.sum(sy_acc, axis=1, keepdims=True)
        sum_ey_y = jnp.sum(eyy_acc, axis=1, keepdims=True)
        sum_ey_u = jnp.sum(eyu_acc, axis=1, keepdims=True)
        s_u = jnp.sum(su_acc, axis=1, keepdims=True)
        lse_y = jnp.log2(s_y) * ln2
        lse_u = jnp.log2(s_u) * ln2
        kl_g = (sum_ey_y - sum_ey_u) / s_y - lse_y + lse_u
        kl_parts.append(jnp.sum(kl_g))
        total = total + ln2 * jnp.sum(f_acc)

        cand_hi = jnp.concatenate(acc_hi, axis=1)
        t_hi = _rank_k(cand_hi, k, largest=True)
        thi_ref[g * _GR:(g + 1) * _GR, :] = t_hi
        chi_ref[g * _GR:(g + 1) * _GR, :] = _coef(cand_hi, t_hi, k, True)
        bad_hi_any = jnp.maximum(
            bad_hi_any, jnp.max(jnp.where(acc_hi[-1] >= t_hi, 1.0, 0.0)))

        cand_lo = jnp.concatenate(acc_lo, axis=1)
        t_lo = _rank_k(cand_lo, k, largest=False)
        tlo_ref[g * _GR:(g + 1) * _GR, :] = t_lo
        clo_ref[g * _GR:(g + 1) * _GR, :] = _coef(cand_lo, t_lo, k, False)
        bad_lo_any = jnp.maximum(
            bad_lo_any, jnp.max(jnp.where(acc_lo[-1] <= t_lo, 1.0, 0.0)))

    @pl.when(bad_hi_any > 0.5)
    def _fb_hi():
        y = yt_ref[...]
        t = _rank_k(_fold_full(y, k, largest=True), k, largest=True)
        thi_ref[...] = t
        chi_ref[...] = _coef(y, t, k, largest=True)

    @pl.when(bad_lo_any > 0.5)
    def _fb_lo():
        y = yt_ref[...]
        t = _rank_k(_fold_full(y, k, largest=False), k, largest=False)
        tlo_ref[...] = t
        clo_ref[...] = _coef(y, t, k, largest=False)

    for g in range(n_groups):
        t_hi = thi_ref[g * _GR:(g + 1) * _GR, :]
        t_lo = tlo_ref[g * _GR:(g + 1) * _GR, :]
        up_s = jnp.zeros((_GR, _LANE), jnp.float32)
        up_e = jnp.zeros((_GR, _LANE), jnp.float32)
        dn_s = jnp.zeros((_GR, _LANE), jnp.float32)
        dn_e = jnp.zeros((_GR, _LANE), jnp.float32)
        for j in range(n_slices):
            y_hi = padded(load(yt_ref, g, j), _NEG_BIG)
            y_lo = y_hi if j < n_full else padded(load(yt_ref, g, j),
                                                  _POS_BIG)
            usl = padded(load(up_ref, g, j), _NEG_BIG)
            dsl = padded(load(dn_ref, g, j), _NEG_BIG)
            up_s = up_s + jnp.where(y_hi > t_hi, usl, 0.0)
            up_e = up_e + jnp.where(y_hi == t_hi, usl, 0.0)
            dn_s = dn_s + jnp.where(y_lo < t_lo, dsl, 0.0)
            dn_e = dn_e + jnp.where(y_lo == t_lo, dsl, 0.0)
        t_up = (jnp.sum(up_s, axis=1, keepdims=True)
                + chi_ref[g * _GR:(g + 1) * _GR, :]
                * jnp.sum(up_e, axis=1, keepdims=True))
        t_dn = (jnp.sum(dn_s, axis=1, keepdims=True)
                + clo_ref[g * _GR:(g + 1) * _GR, :]
                * jnp.sum(dn_e, axis=1, keepdims=True))
        total = total - jnp.float32(_UP_WEIGHT) * jnp.sum(t_up) \
                      - jnp.float32(_DOWN_WEIGHT) * jnp.sum(t_dn)

    for p in kl_parts:
        total = total + jnp.float32(_RANKING_WEIGHT) * p

    @pl.when(i == 0)
    def _init():
        out_ref[0, 0] = total

    @pl.when(i != 0)
    def _acc():
        out_ref[0, 0] += total


def kernel(up_logits, down_logits, y_true, masks):
    del masks
    B, N = up_logits.shape
    R = 64
    assert B % R == 0
    out = pl.pallas_call(
        _body,
        grid=(B // R,),
        in_specs=[pl.BlockSpec((R, N), lambda i: (i, 0))] * 3,
        out_specs=pl.BlockSpec((1, 1), lambda i: (0, 0),
                               memory_space=pltpu.SMEM),
        out_shape=jax.ShapeDtypeStruct((1, 1), jnp.float32),
        scratch_shapes=[pltpu.VMEM((R, 1), jnp.float32)] * 4,
    )(up_logits, down_logits, y_true)
    return (out[0, 0] / jnp.float32(B * N)).astype(jnp.float32)


# speedup vs baseline: 4.5476x; 1.0015x over previous
import jax
import jax.numpy as jnp
from jax.experimental import pallas as pl
from jax.experimental.pallas import tpu as pltpu

_TOP_K = 10
_RANKING_WEIGHT = 0.3
_UP_WEIGHT = 1.0
_DOWN_WEIGHT = 0.5
_LANE = 128
_DEPTH = 4
_GR = 8

_NEG_INF = float("-inf")
_POS_INF = float("inf")
_NEG_BIG = -1e30
_POS_BIG = 1e30

_LOG2E = 1.4426950408889634
_LN2 = 0.6931471805599453


def _rank_k(cand, k, largest):
    sent = _NEG_INF if largest else _POS_INF
    red = (lambda a: jnp.max(a, axis=1, keepdims=True)) if largest else \
          (lambda a: jnp.min(a, axis=1, keepdims=True))
    rows = cand.shape[0]
    kf = jnp.float32(k)
    cum = jnp.zeros((rows, 1), jnp.float32)
    t = jnp.zeros((rows, 1), jnp.float32)
    m = None
    for i in range(k):
        if i == 0:
            m = red(cand)
        else:
            if largest:
                m = red(jnp.where(cand < m, cand, sent))
            else:
                m = red(jnp.where(cand > m, cand, sent))
        c = jnp.sum((cand == m).astype(jnp.float32), axis=1, keepdims=True)
        crossed = jnp.logical_and(cum < kf, cum + c >= kf)
        t = t + jnp.where(crossed, m, 0.0)
        cum = cum + c
    return t


def _coef(vals, t, k, largest):
    strict = (vals > t) if largest else (vals < t)
    eq = vals == t
    cnt_s = jnp.sum(strict.astype(jnp.float32), axis=1, keepdims=True)
    cnt_e = jnp.sum(eq.astype(jnp.float32), axis=1, keepdims=True)
    return jnp.clip((jnp.float32(k) - cnt_s) / jnp.maximum(cnt_e, 1.0),
                    0.0, 1.0)


def _fold_full(y, depth, largest):
    rows, N = y.shape
    n_full = N // _LANE
    sent = _NEG_INF if largest else _POS_INF
    pad_v = _NEG_BIG if largest else _POS_BIG
    acc = [jnp.full((rows, _LANE), sent, jnp.float32) for _ in range(depth)]
    sls = [y[:, j * _LANE:(j + 1) * _LANE] for j in range(n_full)]
    rem = N - n_full * _LANE
    if rem:
        pad = jnp.full((rows, _LANE - rem), pad_v, jnp.float32)
        sls.append(jnp.concatenate([y[:, n_full * _LANE:], pad], axis=1))
    for x in sls:
        for d in range(depth):
            if largest:
                hi = jnp.maximum(acc[d], x)
                x = jnp.minimum(acc[d], x)
            else:
                hi = jnp.minimum(acc[d], x)
                x = jnp.maximum(acc[d], x)
            acc[d] = hi
    return jnp.concatenate(acc, axis=1)


def _body(up_ref, dn_ref, yt_ref, out_ref, thi_ref, tlo_ref,
          chi_ref, clo_ref):
    i = pl.program_id(0)
    R, N = yt_ref.shape
    k = min(_TOP_K, N)
    n_full = N // _LANE
    rem = N - n_full * _LANE
    n_groups = R // _GR
    log2e = jnp.float32(_LOG2E)
    ln2 = jnp.float32(_LN2)

    def load(ref, g, j):
        r0 = g * _GR
        if j < n_full:
            return ref[r0:r0 + _GR, j * _LANE:(j + 1) * _LANE]
        return ref[r0:r0 + _GR, n_full * _LANE:]

    def padded(x, pad_v):
        if x.shape[1] == _LANE:
            return x
        return jnp.concatenate(
            [x, jnp.full((x.shape[0], _LANE - x.shape[1]), pad_v,
                         jnp.float32)], axis=1)

    n_slices = n_full + (1 if rem else 0)

    total = jnp.float32(0.0)
    bad_hi_any = jnp.float32(0.0)
    bad_lo_any = jnp.float32(0.0)
    kl_parts = []

    for g in range(n_groups):
        zeros = jnp.zeros((_GR, _LANE), jnp.float32)
        f_acc = zeros
        sy_acc = zeros
        eyy_acc = zeros
        eyu_acc = zeros
        su_acc = zeros
        acc_hi = [jnp.full((_GR, _LANE), _NEG_INF, jnp.float32)
                  for _ in range(_DEPTH)]
        acc_lo = [jnp.full((_GR, _LANE), _POS_INF, jnp.float32)
                  for _ in range(_DEPTH)]
        def dense(j):
            ysl = padded(load(yt_ref, g, j), _NEG_BIG)
            usl = padded(load(up_ref, g, j), _NEG_BIG)
            dsl = padded(load(dn_ref, g, j), _NEG_BIG)
            nonlocal f_acc, sy_acc, eyy_acc, eyu_acc, su_acc
            p_u = jnp.exp2(usl * log2e)
            p_d = jnp.exp2(dsl * log2e)
            e_y = jnp.exp2(ysl * log2e)
            su_acc = su_acc + p_u
            f_acc = (f_acc + jnp.log2(1.0 + p_u)
                     + jnp.float32(_DOWN_WEIGHT) * jnp.log2(1.0 + p_d))
            sy_acc = sy_acc + e_y
            eyy_acc = eyy_acc + e_y * ysl
            eyu_acc = eyu_acc + e_y * usl
            return ysl

        def insert(acc, x, largest, skip0=False):
            for d in range(1 if skip0 else 0, _DEPTH):
                if largest:
                    keep = jnp.maximum(acc[d], x)
                    x = jnp.minimum(acc[d], x)
                else:
                    keep = jnp.minimum(acc[d], x)
                    x = jnp.maximum(acc[d], x)
                acc[d] = keep

        n_pairs_shared = (n_slices - 2) // 2
        for jp in range(n_pairs_shared):
            y1 = dense(2 * jp)
            y2 = dense(2 * jp + 1)
            ph = jnp.maximum(y1, y2)
            pl_ = jnp.minimum(y1, y2)
            insert(acc_hi, ph, True)
            insert(acc_hi, pl_, True, skip0=True)
            insert(acc_lo, pl_, False)
            insert(acc_lo, ph, False, skip0=True)
        j1 = 2 * n_pairs_shared
        j2 = j1 + 1
        y1 = dense(j1)
        _ = dense(j2)
        y2_hi = padded(load(yt_ref, g, j2), _NEG_BIG)
        y2_lo = (y2_hi if j2 < n_full
                 else padded(load(yt_ref, g, j2), _POS_BIG))
        ph = jnp.maximum(y1, y2_hi)
        pl_ = jnp.minimum(y1, y2_hi)
        insert(acc_hi, ph, True)
        insert(acc_hi, pl_, True, skip0=True)
        ph2 = jnp.maximum(y1, y2_lo)
        pl2 = jnp.minimum(y1, y2_lo)
        insert(acc_lo, pl2, False)
        insert(acc_lo, ph2, False, skip0=True)

        s_y = jnp.sum(sy_acc, axis=1, keepdims=True)
        sum_ey_y = jnp.sum(eyy_acc, axis=1, keepdims=True)
        sum_ey_u = jnp.sum(eyu_acc, axis=1, keepdims=True)
        s_u = jnp.sum(su_acc, axis=1, keepdims=True)
        lse_y = jnp.log2(s_y) * ln2
        lse_u = jnp.log2(s_u) * ln2
        kl_g = (sum_ey_y - sum_ey_u) / s_y - lse_y + lse_u
        kl_parts.append(jnp.sum(kl_g))
        total = total + ln2 * jnp.sum(f_acc)

        cand_hi = jnp.concatenate(acc_hi, axis=1)
        t_hi = _rank_k(cand_hi, k, largest=True)
        thi_ref[g * _GR:(g + 1) * _GR, :] = t_hi
        chi_ref[g * _GR:(g + 1) * _GR, :] = _coef(cand_hi, t_hi, k, True)
        bad_hi_any = jnp.maximum(
            bad_hi_any, jnp.max(jnp.where(acc_hi[-1] >= t_hi, 1.0, 0.0)))

        cand_lo = jnp.concatenate(acc_lo, axis=1)
        t_lo = _rank_k(cand_lo, k, largest=False)
        tlo_ref[g * _GR:(g + 1) * _GR, :] = t_lo
        clo_ref[g * _GR:(g + 1) * _GR, :] = _coef(cand_lo, t_lo, k, False)
        bad_lo_any = jnp.maximum(
            bad_lo_any, jnp.max(jnp.where(acc_lo[-1] <= t_lo, 1.0, 0.0)))

    @pl.when(bad_hi_any > 0.5)
    def _fb_hi():
        y = yt_ref[...]
        t = _rank_k(_fold_full(y, k, largest=True), k, largest=True)
        thi_ref[...] = t
        chi_ref[...] = _coef(y, t, k, largest=True)

    @pl.when(bad_lo_any > 0.5)
    def _fb_lo():
        y = yt_ref[...]
        t = _rank_k(_fold_full(y, k, largest=False), k, largest=False)
        tlo_ref[...] = t
        clo_ref[...] = _coef(y, t, k, largest=False)

    for g in range(n_groups):
        t_hi = thi_ref[g * _GR:(g + 1) * _GR, :]
        t_lo = tlo_ref[g * _GR:(g + 1) * _GR, :]
        up_s = jnp.zeros((_GR, _LANE), jnp.float32)
        up_e = jnp.zeros((_GR, _LANE), jnp.float32)
        dn_s = jnp.zeros((_GR, _LANE), jnp.float32)
        dn_e = jnp.zeros((_GR, _LANE), jnp.float32)
        for j in range(n_slices):
            y_hi = padded(load(yt_ref, g, j), _NEG_BIG)
            y_lo = y_hi if j < n_full else padded(load(yt_ref, g, j),
                                                  _POS_BIG)
            usl = padded(load(up_ref, g, j), _NEG_BIG)
            dsl = padded(load(dn_ref, g, j), _NEG_BIG)
            up_s = up_s + jnp.where(y_hi > t_hi, usl, 0.0)
            up_e = up_e + jnp.where(y_hi == t_hi, usl, 0.0)
            dn_s = dn_s + jnp.where(y_lo < t_lo, dsl, 0.0)
            dn_e = dn_e + jnp.where(y_lo == t_lo, dsl, 0.0)
        t_up = (jnp.sum(up_s, axis=1, keepdims=True)
                + chi_ref[g * _GR:(g + 1) * _GR, :]
                * jnp.sum(up_e, axis=1, keepdims=True))
        t_dn = (jnp.sum(dn_s, axis=1, keepdims=True)
                + clo_ref[g * _GR:(g + 1) * _GR, :]
                * jnp.sum(dn_e, axis=1, keepdims=True))
        total = total - jnp.float32(_UP_WEIGHT) * jnp.sum(t_up) \
                      - jnp.float32(_DOWN_WEIGHT) * jnp.sum(t_dn)

    for p in kl_parts:
        total = total + jnp.float32(_RANKING_WEIGHT) * p

    @pl.when(i == 0)
    def _init():
        out_ref[0, 0] = total

    @pl.when(i != 0)
    def _acc():
        out_ref[0, 0] += total


def kernel(up_logits, down_logits, y_true, masks):
    del masks
    B, N = up_logits.shape
    R = 64
    assert B % R == 0
    out = pl.pallas_call(
        _body,
        grid=(B // R,),
        in_specs=[pl.BlockSpec((R, N), lambda i: (i, 0))] * 3,
        out_specs=pl.BlockSpec((1, 1), lambda i: (0, 0),
                               memory_space=pltpu.SMEM),
        out_shape=jax.ShapeDtypeStruct((1, 1), jnp.float32),
        scratch_shapes=[pltpu.VMEM((R, 1), jnp.float32)] * 4,
    )(up_logits, down_logits, y_true)
    return (out[0, 0] / jnp.float32(B * N)).astype(jnp.float32)


# speedup vs baseline: 4.5627x; 1.0033x over previous
import jax
import jax.numpy as jnp
from jax.experimental import pallas as pl
from jax.experimental.pallas import tpu as pltpu

_TOP_K = 10
_RANKING_WEIGHT = 0.3
_UP_WEIGHT = 1.0
_DOWN_WEIGHT = 0.5
_LANE = 128
_DEPTH = 4
_GR = 8

_NEG_INF = float("-inf")
_POS_INF = float("inf")
_NEG_BIG = -1e30
_POS_BIG = 1e30

_LOG2E = 1.4426950408889634
_LN2 = 0.6931471805599453


def _rank_k_distinct(cand, k, largest):
    sent = _NEG_INF if largest else _POS_INF
    red = (lambda a: jnp.max(a, axis=1, keepdims=True)) if largest else \
          (lambda a: jnp.min(a, axis=1, keepdims=True))
    m = red(cand)
    for _ in range(k - 1):
        if largest:
            m = red(jnp.where(cand < m, cand, sent))
        else:
            m = red(jnp.where(cand > m, cand, sent))
    beyond = (cand >= m) if largest else (cand <= m)
    cnt_ge = jnp.sum(beyond.astype(jnp.float32), axis=1, keepdims=True)
    return m, cnt_ge


def _rank_k(cand, k, largest):
    sent = _NEG_INF if largest else _POS_INF
    red = (lambda a: jnp.max(a, axis=1, keepdims=True)) if largest else \
          (lambda a: jnp.min(a, axis=1, keepdims=True))
    rows = cand.shape[0]
    kf = jnp.float32(k)
    cum = jnp.zeros((rows, 1), jnp.float32)
    t = jnp.zeros((rows, 1), jnp.float32)
    m = None
    for i in range(k):
        if i == 0:
            m = red(cand)
        else:
            if largest:
                m = red(jnp.where(cand < m, cand, sent))
            else:
                m = red(jnp.where(cand > m, cand, sent))
        c = jnp.sum((cand == m).astype(jnp.float32), axis=1, keepdims=True)
        crossed = jnp.logical_and(cum < kf, cum + c >= kf)
        t = t + jnp.where(crossed, m, 0.0)
        cum = cum + c
    return t


def _coef(vals, t, k, largest):
    strict = (vals > t) if largest else (vals < t)
    eq = vals == t
    cnt_s = jnp.sum(strict.astype(jnp.float32), axis=1, keepdims=True)
    cnt_e = jnp.sum(eq.astype(jnp.float32), axis=1, keepdims=True)
    return jnp.clip((jnp.float32(k) - cnt_s) / jnp.maximum(cnt_e, 1.0),
                    0.0, 1.0)


def _fold_full(y, depth, largest):
    rows, N = y.shape
    n_full = N // _LANE
    sent = _NEG_INF if largest else _POS_INF
    pad_v = _NEG_BIG if largest else _POS_BIG
    acc = [jnp.full((rows, _LANE), sent, jnp.float32) for _ in range(depth)]
    sls = [y[:, j * _LANE:(j + 1) * _LANE] for j in range(n_full)]
    rem = N - n_full * _LANE
    if rem:
        pad = jnp.full((rows, _LANE - rem), pad_v, jnp.float32)
        sls.append(jnp.concatenate([y[:, n_full * _LANE:], pad], axis=1))
    for x in sls:
        for d in range(depth):
            if largest:
                hi = jnp.maximum(acc[d], x)
                x = jnp.minimum(acc[d], x)
            else:
                hi = jnp.minimum(acc[d], x)
                x = jnp.maximum(acc[d], x)
            acc[d] = hi
    return jnp.concatenate(acc, axis=1)


def _body(up_ref, dn_ref, yt_ref, out_ref, thi_ref, tlo_ref,
          chi_ref, clo_ref, msum_ref):
    i = pl.program_id(0)
    R, N = yt_ref.shape
    k = min(_TOP_K, N)
    n_full = N // _LANE
    rem = N - n_full * _LANE
    n_groups = R // _GR
    log2e = jnp.float32(_LOG2E)
    ln2 = jnp.float32(_LN2)

    def load(ref, g, j):
        r0 = g * _GR
        if j < n_full:
            return ref[r0:r0 + _GR, j * _LANE:(j + 1) * _LANE]
        return ref[r0:r0 + _GR, n_full * _LANE:]

    def padded(x, pad_v):
        if x.shape[1] == _LANE:
            return x
        return jnp.concatenate(
            [x, jnp.full((x.shape[0], _LANE - x.shape[1]), pad_v,
                         jnp.float32)], axis=1)

    n_slices = n_full + (1 if rem else 0)

    total = jnp.float32(0.0)
    bad_hi_any = jnp.float32(0.0)
    bad_lo_any = jnp.float32(0.0)
    kl_parts = []

    for g in range(n_groups):
        zeros = jnp.zeros((_GR, _LANE), jnp.float32)
        f_acc = zeros
        sy_acc = zeros
        eyy_acc = zeros
        eyu_acc = zeros
        su_acc = zeros
        acc_hi = [jnp.full((_GR, _LANE), _NEG_INF, jnp.float32)
                  for _ in range(_DEPTH)]
        acc_lo = [jnp.full((_GR, _LANE), _POS_INF, jnp.float32)
                  for _ in range(_DEPTH)]
        def dense(j):
            ysl = padded(load(yt_ref, g, j), _NEG_BIG)
            usl = padded(load(up_ref, g, j), _NEG_BIG)
            dsl = padded(load(dn_ref, g, j), _NEG_BIG)
            nonlocal f_acc, sy_acc, eyy_acc, eyu_acc, su_acc
            p_u = jnp.exp2(usl * log2e)
            p_d = jnp.exp2(dsl * log2e)
            e_y = jnp.exp2(ysl * log2e)
            su_acc = su_acc + p_u
            f_acc = (f_acc + jnp.log2(1.0 + p_u)
                     + jnp.float32(_DOWN_WEIGHT) * jnp.log2(1.0 + p_d))
            sy_acc = sy_acc + e_y
            eyy_acc = eyy_acc + e_y * ysl
            eyu_acc = eyu_acc + e_y * usl
            return ysl

        def insert(acc, x, largest, skip=0):
            for d in range(skip, _DEPTH):
                if largest:
                    keep = jnp.maximum(acc[d], x)
                    x = jnp.minimum(acc[d], x)
                else:
                    keep = jnp.minimum(acc[d], x)
                    x = jnp.maximum(acc[d], x)
                acc[d] = keep

        def sort4(y0, y1, y2, y3):
            a = jnp.maximum(y0, y1)
            b = jnp.minimum(y0, y1)
            c = jnp.maximum(y2, y3)
            d = jnp.minimum(y2, y3)
            s0 = jnp.maximum(a, c)
            t1 = jnp.minimum(a, c)
            s3 = jnp.minimum(b, d)
            t2 = jnp.maximum(b, d)
            s1 = jnp.maximum(t1, t2)
            s2 = jnp.minimum(t1, t2)
            return s0, s1, s2, s3

        def insert_sorted(s, largest):
            order = s if largest else s[::-1]
            acc = acc_hi if largest else acc_lo
            for idx, x in enumerate(order):
                insert(acc, x, largest, skip=idx)

        assert n_slices % 4 == 0
        n_batches = n_slices // 4 - 1
        for jb in range(n_batches):
            ys = tuple(dense(4 * jb + t) for t in range(4))
            s = sort4(*ys)
            insert_sorted(s, True)
            insert_sorted(s, False)
        j0 = 4 * n_batches
        ys3 = tuple(dense(j0 + t) for t in range(3))
        _ = dense(j0 + 3)
        tail_hi = padded(load(yt_ref, g, j0 + 3), _NEG_BIG)
        tail_lo = (tail_hi if j0 + 3 < n_full
                   else padded(load(yt_ref, g, j0 + 3), _POS_BIG))
        insert_sorted(sort4(*ys3, tail_hi), True)
        insert_sorted(sort4(*ys3, tail_lo), False)

        s_y = jnp.sum(sy_acc, axis=1, keepdims=True)
        sum_ey_y = jnp.sum(eyy_acc, axis=1, keepdims=True)
        sum_ey_u = jnp.sum(eyu_acc, axis=1, keepdims=True)
        s_u = jnp.sum(su_acc, axis=1, keepdims=True)
        lse_y = jnp.log2(s_y) * ln2
        lse_u = jnp.log2(s_u) * ln2
        kl_g = (sum_ey_y - sum_ey_u) / s_y - lse_y + lse_u
        kl_parts.append(jnp.sum(kl_g))
        total = total + ln2 * jnp.sum(f_acc)

        kf = jnp.float32(k)
        one = jnp.ones((_GR, 1), jnp.float32)
        cand_hi = jnp.concatenate(acc_hi, axis=1)
        t_hi, cge_hi = _rank_k_distinct(cand_hi, k, largest=True)
        thi_ref[g * _GR:(g + 1) * _GR, :] = t_hi
        chi_ref[g * _GR:(g + 1) * _GR, :] = one
        bad_hi_any = jnp.maximum(
            bad_hi_any,
            jnp.maximum(jnp.max(jnp.where(acc_hi[-1] >= t_hi, 1.0, 0.0)),
                        jnp.max(jnp.where(cge_hi != kf, 1.0, 0.0))))

        cand_lo = jnp.concatenate(acc_lo, axis=1)
        t_lo, cge_lo = _rank_k_distinct(cand_lo, k, largest=False)
        tlo_ref[g * _GR:(g + 1) * _GR, :] = t_lo
        clo_ref[g * _GR:(g + 1) * _GR, :] = one
        bad_lo_any = jnp.maximum(
            bad_lo_any,
            jnp.maximum(jnp.max(jnp.where(acc_lo[-1] <= t_lo, 1.0, 0.0)),
                        jnp.max(jnp.where(cge_lo != kf, 1.0, 0.0))))

    @pl.when(bad_hi_any > 0.5)
    def _fb_hi():
        y = yt_ref[...]
        t = _rank_k(_fold_full(y, k, largest=True), k, largest=True)
        thi_ref[...] = t
        chi_ref[...] = _coef(y, t, k, largest=True)

    @pl.when(bad_lo_any > 0.5)
    def _fb_lo():
        y = yt_ref[...]
        t = _rank_k(_fold_full(y, k, largest=False), k, largest=False)
        tlo_ref[...] = t
        clo_ref[...] = _coef(y, t, k, largest=False)

    allone = jnp.logical_and(jnp.min(chi_ref[...]) >= 1.0,
                             jnp.min(clo_ref[...]) >= 1.0)

    @pl.when(allone)
    def _sweep2_fast():
        tot = jnp.float32(0.0)
        for g in range(n_groups):
            t_hi = thi_ref[g * _GR:(g + 1) * _GR, :]
            t_lo = tlo_ref[g * _GR:(g + 1) * _GR, :]
            up_s = jnp.zeros((_GR, _LANE), jnp.float32)
            dn_s = jnp.zeros((_GR, _LANE), jnp.float32)
            for j in range(n_slices):
                y_hi = padded(load(yt_ref, g, j), _NEG_BIG)
                y_lo = y_hi if j < n_full else padded(load(yt_ref, g, j),
                                                      _POS_BIG)
                usl = padded(load(up_ref, g, j), _NEG_BIG)
                dsl = padded(load(dn_ref, g, j), _NEG_BIG)
                up_s = up_s + jnp.where(y_hi >= t_hi, usl, 0.0)
                dn_s = dn_s + jnp.where(y_lo <= t_lo, dsl, 0.0)
            tot = tot - jnp.float32(_UP_WEIGHT) * jnp.sum(up_s) \
                      - jnp.float32(_DOWN_WEIGHT) * jnp.sum(dn_s)
        msum_ref[0, 0] = tot

    @pl.when(jnp.logical_not(allone))
    def _sweep2_full():
        tot = jnp.float32(0.0)
        for g in range(n_groups):
            t_hi = thi_ref[g * _GR:(g + 1) * _GR, :]
            t_lo = tlo_ref[g * _GR:(g + 1) * _GR, :]
            up_s = jnp.zeros((_GR, _LANE), jnp.float32)
            up_e = jnp.zeros((_GR, _LANE), jnp.float32)
            dn_s = jnp.zeros((_GR, _LANE), jnp.float32)
            dn_e = jnp.zeros((_GR, _LANE), jnp.float32)
            for j in range(n_slices):
                y_hi = padded(load(yt_ref, g, j), _NEG_BIG)
                y_lo = y_hi if j < n_full else padded(load(yt_ref, g, j),
                                                      _POS_BIG)
                usl = padded(load(up_ref, g, j), _NEG_BIG)
                dsl = padded(load(dn_ref, g, j), _NEG_BIG)
                up_s = up_s + jnp.where(y_hi > t_hi, usl, 0.0)
                up_e = up_e + jnp.where(y_hi == t_hi, usl, 0.0)
                dn_s = dn_s + jnp.where(y_lo < t_lo, dsl, 0.0)
                dn_e = dn_e + jnp.where(y_lo == t_lo, dsl, 0.0)
            t_up = (jnp.sum(up_s, axis=1, keepdims=True)
                    + chi_ref[g * _GR:(g + 1) * _GR, :]
                    * jnp.sum(up_e, axis=1, keepdims=True))
            t_dn = (jnp.sum(dn_s, axis=1, keepdims=True)
                    + clo_ref[g * _GR:(g + 1) * _GR, :]
                    * jnp.sum(dn_e, axis=1, keepdims=True))
            tot = tot - jnp.float32(_UP_WEIGHT) * jnp.sum(t_up) \
                      - jnp.float32(_DOWN_WEIGHT) * jnp.sum(t_dn)
        msum_ref[0, 0] = tot

    total = total + msum_ref[0, 0]

    for p in kl_parts:
        total = total + jnp.float32(_RANKING_WEIGHT) * p

    @pl.when(i == 0)
    def _init():
        out_ref[0, 0] = total

    @pl.when(i != 0)
    def _acc():
        out_ref[0, 0] += total


def kernel(up_logits, down_logits, y_true, masks):
    del masks
    B, N = up_logits.shape
    R = 64
    assert B % R == 0
    out = pl.pallas_call(
        _body,
        grid=(B // R,),
        in_specs=[pl.BlockSpec((R, N), lambda i: (i, 0))] * 3,
        out_specs=pl.BlockSpec((1, 1), lambda i: (0, 0),
                               memory_space=pltpu.SMEM),
        out_shape=jax.ShapeDtypeStruct((1, 1), jnp.float32),
        scratch_shapes=[pltpu.VMEM((R, 1), jnp.float32)] * 4
        + [pltpu.SMEM((1, 1), jnp.float32)],
    )(up_logits, down_logits, y_true)
    return (out[0, 0] / jnp.float32(B * N)).astype(jnp.float32)
